# baseline (speedup 1.0000x reference)
import functools

import jax
import jax.numpy as jnp
from jax import lax
from jax.experimental import pallas as pl
from jax.experimental.pallas import tpu as pltpu
from jax.experimental.pallas import tpu_sc as plsc


def _labels_body(C, logits_ref, gate_ref, out_ref):
    x = logits_ref[...]
    m = jnp.max(x, axis=1, keepdims=True)
    col = lax.broadcasted_iota(jnp.int32, x.shape, 1)
    idx = jnp.min(jnp.where(x == m, col, C), axis=1)
    g = gate_ref[0, 0, :]
    out_ref[0, 0, :] = jnp.where(g > 0, idx, C).reshape(1, 1, -1)[0, 0, :]


def _compute_labels(logits, gate3, nt, blk, row_off=0):
    C = logits.shape[1]
    nb = nt // blk
    ob = row_off // blk
    return pl.pallas_call(
        functools.partial(_labels_body, C),
        grid=(nb,),
        in_specs=[
            pl.BlockSpec((blk, C), lambda i: (i + ob, 0)),
            pl.BlockSpec((1, 1, blk), lambda i: (i, 0, 0)),
        ],
        out_specs=pl.BlockSpec((1, 1, blk), lambda i: (i, 0, 0)),
        out_shape=jax.ShapeDtypeStruct((nb, 1, blk), jnp.int32),
    )(logits, gate3)


def _make_segment_sum(nt, D, CP, chunk, row_off=0):
    info = plsc.get_sparse_core_info()
    nc, ns = info.num_cores, info.num_subcores
    rows_per_tile = nt // (nc * ns)
    n_chunks = rows_per_tile // chunk
    lrows = chunk // 128

    mesh = plsc.VectorSubcoreMesh(core_axis_name="c", subcore_axis_name="s")

    @functools.partial(
        pl.kernel,
        mesh=mesh,
        out_type=[
            jax.ShapeDtypeStruct((nc, CP, D), jnp.float32),
            jax.ShapeDtypeStruct((nc, CP, D), jnp.float32),
        ],
        scratch_types=[
            [pltpu.VMEM((128,), jnp.int32) for _ in range(lrows)],
            pltpu.VMEM((chunk, D), jnp.float32),
            pltpu.VMEM((chunk, D), jnp.float32),
            pltpu.VMEM_SHARED((CP, D), jnp.float32),
            pltpu.VMEM_SHARED((CP, D), jnp.float32),
        ],
    )
    def seg(lbl_hbm, img_hbm, zsum_hbm, zcnt_hbm, ones_hbm,
            sums_out, cnts_out, lbl_vs, img_v, ones_v, sums_sh, cnts_sh):
        cid = lax.axis_index("c")
        sid = lax.axis_index("s")

        @pl.when(sid == 0)
        def _():
            pltpu.sync_copy(zsum_hbm, sums_sh)
            pltpu.sync_copy(zcnt_hbm, cnts_sh)

        pltpu.sync_copy(ones_hbm, ones_v)
        plsc.subcore_barrier()

        lbase = (cid * ns + sid) * rows_per_tile
        for j in range(n_chunks):
            l0 = lbase + j * chunk
            r0 = pl.multiple_of(row_off + l0, chunk)
            pltpu.sync_copy(img_hbm.at[pl.ds(r0, chunk)], img_v)
            for k in range(lrows):
                rk = pl.multiple_of(l0 + k * 128, 128)
                pltpu.sync_copy(lbl_hbm.at[pl.ds(rk, 128)], lbl_vs[k])
            for k in range(lrows):
                src = img_v.at[pl.ds(k * 128, 128)]
                pltpu.sync_copy(src, sums_sh.at[lbl_vs[k]], add=True)
                pltpu.sync_copy(ones_v.at[pl.ds(k * 128, 128)],
                                cnts_sh.at[lbl_vs[k]], add=True)

        plsc.subcore_barrier()

        @pl.when(sid == 0)
        def _():
            pltpu.sync_copy(sums_sh, sums_out.at[cid])
            pltpu.sync_copy(cnts_sh, cnts_out.at[cid])

    return seg


def _final_body(C, sa_ref, sb_ref, ca_ref, cb_ref, text_ref, out_ref):
    s = sa_ref[0] + sa_ref[1] + sb_ref[0] + sb_ref[1]
    cnt = (ca_ref[0, :, 0] + ca_ref[1, :, 0]
           + cb_ref[0, :, 0] + cb_ref[1, :, 0])
    CP = s.shape[0]
    rows = lax.broadcasted_iota(jnp.int32, (CP,), 0)
    valid = (cnt > 0.0) & (rows < C)
    safe = jnp.where(cnt > 0.0, cnt, 1.0)
    means = s / safe[:, None]
    d = jnp.sum(means * text_ref[...], axis=1)
    num_present = jnp.sum(valid.astype(jnp.float32))
    loss = jnp.sum(jnp.where(valid, d, 0.0)) / jnp.maximum(num_present, 1.0)
    out_ref[...] = loss.reshape(1, 1)


def _finalize(sumsA, sumsB, cntsA, cntsB, text_pad, C):
    return pl.pallas_call(
        functools.partial(_final_body, C),
        out_shape=jax.ShapeDtypeStruct((1, 1), jnp.float32),
    )(sumsA, sumsB, cntsA, cntsB, text_pad)


@jax.jit
def kernel(logits, img_feats, text_norm_feats, gate_mask):
    N, C = logits.shape
    D = img_feats.shape[1]
    CP = ((C + 1 + 15) // 16) * 16
    BLK = 2048
    CHUNK = 256

    gate_i32 = gate_mask.astype(jnp.int32)

    zsum = jnp.zeros((CP, D), jnp.float32)
    zcnt = jnp.zeros((CP, D), jnp.float32)
    ones256 = jnp.ones((CHUNK, D), jnp.float32)

    H = N // 2
    gate3a = gate_i32[:H].reshape(H // BLK, 1, BLK)
    gate3b = gate_i32[H:].reshape(H // BLK, 1, BLK)
    la = _compute_labels(logits, gate3a, H, BLK).reshape(H)
    segA = _make_segment_sum(H, D, CP, CHUNK)
    sumsA, cntsA = segA(la, img_feats, zsum, zcnt, ones256)
    lb = _compute_labels(logits, gate3b, H, BLK, row_off=H).reshape(H)
    segB = _make_segment_sum(H, D, CP, CHUNK, row_off=H)
    sumsB, cntsB = segB(lb, img_feats, zsum, zcnt, ones256)

    text_pad = jnp.pad(text_norm_feats, ((0, CP - C), (0, 0)))
    loss = _finalize(sumsA, sumsB, cntsA, cntsB, text_pad, C)
    return loss[0, 0]
